# baseline (speedup 1.0000x reference)
import jax
import jax.numpy as jnp
from jax.experimental import pallas as pl
from jax.experimental.pallas import tpu as pltpu

_L = 50
_BR = 6400


def _pe_add_kernel(x_ref, pe_ref, o_ref, pos_ref):
    @pl.when(pl.program_id(0) == 0)
    def _fill():
        for j in range(_BR // _L):
            pos_ref[j * _L:(j + 1) * _L, :] = pe_ref[:_L, :]

    o_ref[...] = x_ref[...] + pos_ref[...]


def kernel(x, pe):
    B, L, D = x.shape
    R = B * L
    x2 = x.reshape(R, D)
    grid = (R // _BR,)
    out = pl.pallas_call(
        _pe_add_kernel,
        grid=grid,
        in_specs=[
            pl.BlockSpec((_BR, D), lambda i: (i, 0)),
            pl.BlockSpec(pe.shape, lambda i: (0, 0)),
        ],
        out_specs=pl.BlockSpec((_BR, D), lambda i: (i, 0)),
        out_shape=jax.ShapeDtypeStruct((R, D), x.dtype),
        scratch_shapes=[pltpu.VMEM((_BR, D), x.dtype)],
    )(x2, pe)
    return out.reshape(B, L, D)


# speedup vs baseline: 1.0013x; 1.0013x over previous
import jax
import jax.numpy as jnp
from jax.experimental import pallas as pl
from jax.experimental.pallas import tpu as pltpu

_L = 50
_P = 200
_BB = 128


def _pe_add_kernel(x_ref, pe_ref, o_ref, pos_ref):
    @pl.when(pl.program_id(0) == 0)
    def _fill():
        for j in range(_P // _L):
            pos_ref[j * _L:(j + 1) * _L, :] = pe_ref[:_L, :]

    o_ref[...] = x_ref[...] + pos_ref[...][None, :, :]


def kernel(x, pe):
    B, L, D = x.shape
    n_per = (B * L) // _P
    x3 = x.reshape(n_per, _P, D)
    grid = (n_per // _BB,)
    out = pl.pallas_call(
        _pe_add_kernel,
        grid=grid,
        in_specs=[
            pl.BlockSpec((_BB, _P, D), lambda i: (i, 0, 0)),
            pl.BlockSpec(pe.shape, lambda i: (0, 0)),
        ],
        out_specs=pl.BlockSpec((_BB, _P, D), lambda i: (i, 0, 0)),
        out_shape=jax.ShapeDtypeStruct((n_per, _P, D), x.dtype),
        scratch_shapes=[pltpu.VMEM((_P, D), x.dtype)],
    )(x3, pe)
    return out.reshape(B, L, D)


# speedup vs baseline: 1.9607x; 1.9581x over previous
import jax
import jax.numpy as jnp
from jax.experimental import pallas as pl
from jax.experimental.pallas import tpu as pltpu

_BB = 256


def _pe_add_kernel(x_ref, pe_ref, o_ref):
    L = x_ref.shape[1]
    o_ref[...] = x_ref[...] + pe_ref[:L, :][None, :, :]


def kernel(x, pe):
    B, L, D = x.shape
    grid = (B // _BB,)
    return pl.pallas_call(
        _pe_add_kernel,
        grid=grid,
        in_specs=[
            pl.BlockSpec((_BB, L, D), lambda i: (i, 0, 0)),
            pl.BlockSpec(pe.shape, lambda i: (0, 0)),
        ],
        out_specs=pl.BlockSpec((_BB, L, D), lambda i: (i, 0, 0)),
        out_shape=jax.ShapeDtypeStruct((B, L, D), x.dtype),
        compiler_params=pltpu.CompilerParams(
            dimension_semantics=("parallel",),
        ),
    )(x, pe)
